# Initial kernel scaffold; baseline (speedup 1.0000x reference)
import functools

import jax
import jax.numpy as jnp
from jax import lax
from jax.experimental import pallas as pl
from jax.experimental.pallas import tpu as pltpu
from jax.experimental.pallas import tpu_sc as plsc

NC = 2
NS = 16
CH = 128
N = 50000
NP = 50176
RPT = NP // NS
NG = 256
HH = 32
H = 2 * HH
R = 2000
NB = N // R


def _sc_mesh():
    return plsc.VectorSubcoreMesh(
        core_axis_name="c", subcore_axis_name="s", num_cores=NC, num_subcores=NS
    )


def _seg_sum(table_flat, idx_g, idx_s, zrows):
    nch = idx_s.shape[0]
    cpt = nch // NS

    def body(table, idxg, idxs, zr, out, acc, gbuf, sbuf, rows, sem):
        c = lax.axis_index("c")
        s = lax.axis_index("s")
        pltpu.sync_copy(zr, acc.at[pl.ds(s * RPT, RPT)])
        plsc.subcore_barrier()

        def step(k, carry):
            ch = s * cpt + k
            pltpu.sync_copy(idxg.at[c, ch], gbuf)
            pltpu.sync_copy(idxs.at[ch], sbuf)
            pltpu.async_copy(table.at[gbuf], rows, sem).wait()
            pltpu.sync_copy(rows, acc.at[sbuf], add=True)
            return carry

        lax.fori_loop(0, cpt, step, 0)
        plsc.subcore_barrier()
        pltpu.sync_copy(acc.at[pl.ds(s * RPT, RPT)], out.at[c, pl.ds(s * RPT, RPT)])

    return pl.kernel(
        body,
        out_type=jax.ShapeDtypeStruct((NC, NP, HH), jnp.float32),
        mesh=_sc_mesh(),
        scratch_types=[
            pltpu.VMEM_SHARED((NP, HH), jnp.float32),
            pltpu.VMEM((CH,), jnp.int32),
            pltpu.VMEM((CH,), jnp.int32),
            pltpu.VMEM((CH, HH), jnp.float32),
            pltpu.SemaphoreType.DMA,
        ],
    )(table_flat, idx_g, idx_s, zrows)


def _seg_count(idx_s2, ones_rows, zrows):
    nch = idx_s2.shape[1]
    cpt = nch // NS

    def body(idxs2, ones_h, zr, out, acc, sbuf, onesv):
        c = lax.axis_index("c")
        s = lax.axis_index("s")
        pltpu.sync_copy(zr, acc.at[pl.ds(s * RPT, RPT)])
        pltpu.sync_copy(ones_h, onesv)
        plsc.subcore_barrier()

        def step(k, carry):
            ch = s * cpt + k
            pltpu.sync_copy(idxs2.at[c, ch], sbuf)
            pltpu.sync_copy(onesv, acc.at[sbuf], add=True)
            return carry

        lax.fori_loop(0, cpt, step, 0)
        plsc.subcore_barrier()
        pltpu.sync_copy(acc.at[pl.ds(s * RPT, RPT)], out.at[c, pl.ds(s * RPT, RPT)])

    return pl.kernel(
        body,
        out_type=jax.ShapeDtypeStruct((NC, NP, 16), jnp.float32),
        mesh=_sc_mesh(),
        scratch_types=[
            pltpu.VMEM_SHARED((NP, 16), jnp.float32),
            pltpu.VMEM((CH,), jnp.int32),
            pltpu.VMEM((CH, 16), jnp.float32),
            pltpu.SemaphoreType.DMA,
        ],
    )(idx_s2, ones_rows, zrows)


def _embed(x, WembT, bemb):
    din = x.shape[1]

    def body(x_ref, w_ref, b_ref, o_ref):
        h = jnp.dot(x_ref[...], w_ref[...], preferred_element_type=jnp.float32)
        h = h + b_ref[...]
        o_ref[0] = h[:, :HH]
        o_ref[1] = h[:, HH:]

    return pl.pallas_call(
        body,
        grid=(NB,),
        in_specs=[
            pl.BlockSpec((R, din), lambda i: (i, 0)),
            pl.BlockSpec((din, H), lambda i: (0, 0)),
            pl.BlockSpec((1, H), lambda i: (0, 0)),
        ],
        out_specs=pl.BlockSpec((NC, R, HH), lambda i: (0, i, 0)),
        out_shape=jax.ShapeDtypeStruct((NC, NP, HH), jnp.float32),
    )(x, WembT, bemb)


def _scale_rows(msum, cnt):
    def body(m_ref, c_ref, o_ref):
        inv = 1.0 / jnp.maximum(c_ref[...], 1.0)
        o_ref[...] = m_ref[...] * jnp.reshape(inv, (1, R, 1))

    return pl.pallas_call(
        body,
        grid=(NB,),
        in_specs=[
            pl.BlockSpec((NC, R, HH), lambda i: (0, i, 0)),
            pl.BlockSpec((R, 1), lambda i: (i, 0)),
        ],
        out_specs=pl.BlockSpec((NC, R, HH), lambda i: (0, i, 0)),
        out_shape=jax.ShapeDtypeStruct((NC, NP, HH), jnp.float32),
    )(msum, cnt)


def _layer(h2, a2, g2, cnt_n, WfT, bf, WcT, bc):
    def body(h_ref, a_ref, g_ref, c_ref, wf_ref, bf_ref, wc_ref, bc_ref, o_ref):
        h64 = jnp.concatenate([h_ref[0], h_ref[1]], axis=-1)
        a64 = jnp.concatenate([a_ref[0], a_ref[1]], axis=-1)
        g64 = jnp.concatenate([g_ref[0], g_ref[1]], axis=-1)
        cn = c_ref[...]
        occ = (cn > 0.0).astype(jnp.float32)
        inv = 1.0 / jnp.maximum(cn, 1.0)
        zn = jnp.concatenate([h64 * occ, a64 * inv, g64 * inv], axis=-1)
        zf = jnp.dot(zn, wf_ref[...], preferred_element_type=jnp.float32) + bf_ref[...]
        zc = jnp.dot(zn, wc_ref[...], preferred_element_type=jnp.float32) + bc_ref[...]
        outv = jax.nn.sigmoid(zf) * jax.nn.softplus(zc)
        hn = jax.nn.softplus(outv + h64)
        o_ref[0] = hn[:, :HH]
        o_ref[1] = hn[:, HH:]

    full = lambda shape: pl.BlockSpec(shape, lambda i: tuple(0 for _ in shape))
    return pl.pallas_call(
        body,
        grid=(NB,),
        in_specs=[
            pl.BlockSpec((NC, R, HH), lambda i: (0, i, 0)),
            pl.BlockSpec((NC, R, HH), lambda i: (0, i, 0)),
            pl.BlockSpec((NC, R, HH), lambda i: (0, i, 0)),
            pl.BlockSpec((R, 1), lambda i: (i, 0)),
            full((3 * H, H)),
            full((1, H)),
            full((3 * H, H)),
            full((1, H)),
        ],
        out_specs=pl.BlockSpec((NC, R, HH), lambda i: (0, i, 0)),
        out_shape=jax.ShapeDtypeStruct((NC, NP, HH), jnp.float32),
    )(h2, a2, g2, cnt_n, WfT, bf, WcT, bc)


def _pool_head(h2, batch_col, WprojT, bproj, WoutT, bout):
    hout = WprojT.shape[1]

    def body(h_ref, b_ref, wp_ref, bp_ref, wo_ref, bo_ref, o_ref, acc, cnt):
        i = pl.program_id(0)

        @pl.when(i == 0)
        def _():
            acc[...] = jnp.zeros_like(acc)
            cnt[...] = jnp.zeros_like(cnt)

        h64 = jnp.concatenate([h_ref[0], h_ref[1]], axis=-1)
        oh = (b_ref[...] == lax.broadcasted_iota(jnp.int32, (1, NG), 1))
        oh = oh.astype(jnp.float32)
        dn = (((0,), (0,)), ((), ()))
        acc[...] += lax.dot_general(oh, h64, dn, preferred_element_type=jnp.float32)
        cnt[...] += lax.dot_general(
            oh, jnp.ones((R, 1), jnp.float32), dn, preferred_element_type=jnp.float32
        )

        @pl.when(i == NB - 1)
        def _():
            pooled = acc[...] / jnp.maximum(cnt[...], 1.0)
            pp = jnp.dot(pooled, wp_ref[...], preferred_element_type=jnp.float32)
            pp = jax.nn.softplus(pp + bp_ref[...])
            o_ref[...] = (
                jnp.dot(pp, wo_ref[...], preferred_element_type=jnp.float32)
                + bo_ref[...]
            )

    full = lambda shape: pl.BlockSpec(shape, lambda i: tuple(0 for _ in shape))
    return pl.pallas_call(
        body,
        grid=(NB,),
        in_specs=[
            pl.BlockSpec((NC, R, HH), lambda i: (0, i, 0)),
            pl.BlockSpec((R, 1), lambda i: (i, 0)),
            full((H, hout)),
            full((1, hout)),
            full((hout, 1)),
            full((1, 1)),
        ],
        out_specs=pl.BlockSpec((NG, 1), lambda i: (0, 0)),
        out_shape=jax.ShapeDtypeStruct((NG, 1), jnp.float32),
        scratch_shapes=[
            pltpu.VMEM((NG, H), jnp.float32),
            pltpu.VMEM((NG, 1), jnp.float32),
        ],
    )(h2, batch_col, WprojT, bproj, WoutT, bout)


def _gate_weightT(W):
    Wt = W.T.astype(jnp.float32)
    return jnp.concatenate(
        [Wt[: H + 35], jnp.zeros((HH - 3, H), jnp.float32), Wt[H + 35 :]], axis=0
    )


def kernel(x, hyperedge_index, hyperedge_attr, batch, Wemb, bemb, Wf, bf, Wc, bc,
           Wproj, bproj, Wout, bout):
    f32 = jnp.float32
    src = hyperedge_index[0].astype(jnp.int32)
    hid = hyperedge_index[1].astype(jnp.int32)
    e = src.shape[0]
    wave = NS * CH
    ep = -(-e // wave) * wave
    nch = ep // CH
    pad = ep - e

    zi = jnp.zeros((pad,), jnp.int32)
    di = jnp.full((pad,), N, jnp.int32)
    srcg = jnp.concatenate([src, zi])
    hidg = jnp.concatenate([hid, zi])
    srcs = jnp.concatenate([src, di]).reshape(nch, CH)
    hids = jnp.concatenate([hid, di]).reshape(nch, CH)
    idx_src_g = jnp.stack([srcg, srcg + NP]).reshape(NC, nch, CH)
    idx_hid_g = jnp.stack([hidg, hidg + NP]).reshape(NC, nch, CH)
    idx_cnt = jnp.stack([srcs, hids])

    zrows = jnp.zeros((RPT, HH), f32)
    zrows16 = jnp.zeros((RPT, 16), f32)
    ones_rows = jnp.ones((CH, 16), f32)

    ap = jnp.pad(hyperedge_attr.astype(f32),
                 ((0, NP - N), (0, H - hyperedge_attr.shape[1])))
    attr_flat = jnp.stack([ap[:, :HH], ap[:, HH:]]).reshape(NC * NP, HH)

    counts = _seg_count(idx_cnt, ones_rows, zrows16)
    cnt_n = counts[0, :, :1]
    cnt_h = counts[1, :, :1]

    att2 = _seg_sum(attr_flat, idx_hid_g, srcs, zrows)

    h2 = _embed(x.astype(f32), Wemb.T.astype(f32), bemb.astype(f32).reshape(1, H))

    n_layers = Wf.shape[0]
    for l in range(n_layers):
        msum2 = _seg_sum(h2.reshape(NC * NP, HH), idx_src_g, hids, zrows)
        m2 = _scale_rows(msum2, cnt_h)
        gsum2 = _seg_sum(m2.reshape(NC * NP, HH), idx_hid_g, srcs, zrows)
        h2 = _layer(h2, att2, gsum2, cnt_n,
                    _gate_weightT(Wf[l]), bf[l].astype(f32).reshape(1, H),
                    _gate_weightT(Wc[l]), bc[l].astype(f32).reshape(1, H))

    return _pool_head(
        h2,
        batch.astype(jnp.int32).reshape(N, 1),
        Wproj.T.astype(f32),
        bproj.astype(f32).reshape(1, -1),
        Wout.T.astype(f32),
        bout.astype(f32).reshape(1, 1),
    )


# speedup vs baseline: 3.8951x; 3.8951x over previous
import functools

import jax
import jax.numpy as jnp
from jax import lax
from jax.experimental import pallas as pl
from jax.experimental.pallas import tpu as pltpu
from jax.experimental.pallas import tpu_sc as plsc

NC = 2
NS = 16
CH = 128
N = 50000
NP = 50176
RPT = NP // NS
NG = 256
HH = 32
H = 2 * HH
R = 2000
NB = N // R


def _sc_mesh():
    return plsc.VectorSubcoreMesh(
        core_axis_name="c", subcore_axis_name="s", num_cores=NC, num_subcores=NS
    )


def _seg_sum(table_flat, idx_g, idx_s, zrows):
    nch = idx_s.shape[0]
    cpt = nch // NS

    def body(table, idxg, idxs, zr, out, acc, gbuf, sbuf, rows, sem):
        c = lax.axis_index("c")
        s = lax.axis_index("s")
        pltpu.sync_copy(zr, acc.at[pl.ds(s * RPT, RPT)])
        plsc.subcore_barrier()

        def step(k, carry):
            ch = s * cpt + k
            pltpu.sync_copy(idxg.at[c, ch], gbuf)
            pltpu.sync_copy(idxs.at[ch], sbuf)
            pltpu.async_copy(table.at[gbuf], rows, sem).wait()
            pltpu.sync_copy(rows, acc.at[sbuf], add=True)
            return carry

        lax.fori_loop(0, cpt, step, 0)
        plsc.subcore_barrier()
        pltpu.sync_copy(acc.at[pl.ds(s * RPT, RPT)], out.at[c, pl.ds(s * RPT, RPT)])

    return pl.kernel(
        body,
        out_type=jax.ShapeDtypeStruct((NC, NP, HH), jnp.float32),
        mesh=_sc_mesh(),
        scratch_types=[
            pltpu.VMEM_SHARED((NP, HH), jnp.float32),
            pltpu.VMEM((CH,), jnp.int32),
            pltpu.VMEM((CH,), jnp.int32),
            pltpu.VMEM((CH, HH), jnp.float32),
            pltpu.SemaphoreType.DMA,
        ],
        compiler_params=pltpu.CompilerParams(use_tc_tiling_on_sc=False),
    )(table_flat, idx_g, idx_s, zrows)


def _seg_count(idx_s2, ones_rows, zrows):
    nch = idx_s2.shape[1]
    cpt = nch // NS

    def body(idxs2, ones_h, zr, out, acc, sbuf, onesv):
        c = lax.axis_index("c")
        s = lax.axis_index("s")
        pltpu.sync_copy(zr, acc.at[pl.ds(s * RPT, RPT)])
        pltpu.sync_copy(ones_h, onesv)
        plsc.subcore_barrier()

        def step(k, carry):
            ch = s * cpt + k
            pltpu.sync_copy(idxs2.at[c, ch], sbuf)
            pltpu.sync_copy(onesv, acc.at[sbuf], add=True)
            return carry

        lax.fori_loop(0, cpt, step, 0)
        plsc.subcore_barrier()
        pltpu.sync_copy(acc.at[pl.ds(s * RPT, RPT)], out.at[c, pl.ds(s * RPT, RPT)])

    return pl.kernel(
        body,
        out_type=jax.ShapeDtypeStruct((NC, NP, 16), jnp.float32),
        mesh=_sc_mesh(),
        scratch_types=[
            pltpu.VMEM_SHARED((NP, 16), jnp.float32),
            pltpu.VMEM((CH,), jnp.int32),
            pltpu.VMEM((CH, 16), jnp.float32),
        ],
        compiler_params=pltpu.CompilerParams(use_tc_tiling_on_sc=False),
    )(idx_s2, ones_rows, zrows)


def _embed(x, WembT, bemb):
    din = x.shape[1]

    def body(x_ref, w_ref, b_ref, o_ref):
        h = jnp.dot(x_ref[...], w_ref[...], preferred_element_type=jnp.float32)
        h = h + b_ref[...]
        o_ref[0] = h[:, :HH]
        o_ref[1] = h[:, HH:]

    return pl.pallas_call(
        body,
        grid=(NB,),
        in_specs=[
            pl.BlockSpec((R, din), lambda i: (i, 0)),
            pl.BlockSpec((din, H), lambda i: (0, 0)),
            pl.BlockSpec((1, H), lambda i: (0, 0)),
        ],
        out_specs=pl.BlockSpec((NC, R, HH), lambda i: (0, i, 0)),
        out_shape=jax.ShapeDtypeStruct((NC, NP, HH), jnp.float32),
    )(x, WembT, bemb)


def _scale_rows(msum, cnt):
    def body(m_ref, c_ref, o_ref):
        inv = 1.0 / jnp.maximum(c_ref[...], 1.0)
        o_ref[...] = m_ref[...] * jnp.reshape(inv, (1, R, 1))

    return pl.pallas_call(
        body,
        grid=(NB,),
        in_specs=[
            pl.BlockSpec((NC, R, HH), lambda i: (0, i, 0)),
            pl.BlockSpec((R, 1), lambda i: (i, 0)),
        ],
        out_specs=pl.BlockSpec((NC, R, HH), lambda i: (0, i, 0)),
        out_shape=jax.ShapeDtypeStruct((NC, NP, HH), jnp.float32),
    )(msum, cnt)


def _layer(h2, a2, g2, cnt_n, WfT, bf, WcT, bc):
    def body(h_ref, a_ref, g_ref, c_ref, wf_ref, bf_ref, wc_ref, bc_ref, o_ref):
        h64 = jnp.concatenate([h_ref[0], h_ref[1]], axis=-1)
        a64 = jnp.concatenate([a_ref[0], a_ref[1]], axis=-1)
        g64 = jnp.concatenate([g_ref[0], g_ref[1]], axis=-1)
        cn = c_ref[...]
        occ = (cn > 0.0).astype(jnp.float32)
        inv = 1.0 / jnp.maximum(cn, 1.0)
        zn = jnp.concatenate([h64 * occ, a64 * inv, g64 * inv], axis=-1)
        zf = jnp.dot(zn, wf_ref[...], preferred_element_type=jnp.float32) + bf_ref[...]
        zc = jnp.dot(zn, wc_ref[...], preferred_element_type=jnp.float32) + bc_ref[...]
        outv = jax.nn.sigmoid(zf) * jax.nn.softplus(zc)
        hn = jax.nn.softplus(outv + h64)
        o_ref[0] = hn[:, :HH]
        o_ref[1] = hn[:, HH:]

    full = lambda shape: pl.BlockSpec(shape, lambda i: tuple(0 for _ in shape))
    return pl.pallas_call(
        body,
        grid=(NB,),
        in_specs=[
            pl.BlockSpec((NC, R, HH), lambda i: (0, i, 0)),
            pl.BlockSpec((NC, R, HH), lambda i: (0, i, 0)),
            pl.BlockSpec((NC, R, HH), lambda i: (0, i, 0)),
            pl.BlockSpec((R, 1), lambda i: (i, 0)),
            full((3 * H, H)),
            full((1, H)),
            full((3 * H, H)),
            full((1, H)),
        ],
        out_specs=pl.BlockSpec((NC, R, HH), lambda i: (0, i, 0)),
        out_shape=jax.ShapeDtypeStruct((NC, NP, HH), jnp.float32),
    )(h2, a2, g2, cnt_n, WfT, bf, WcT, bc)


def _pool_head(h2, batch_col, WprojT, bproj, WoutT, bout):
    hout = WprojT.shape[1]

    def body(h_ref, b_ref, wp_ref, bp_ref, wo_ref, bo_ref, o_ref, acc, cnt):
        i = pl.program_id(0)

        @pl.when(i == 0)
        def _():
            acc[...] = jnp.zeros_like(acc)
            cnt[...] = jnp.zeros_like(cnt)

        h64 = jnp.concatenate([h_ref[0], h_ref[1]], axis=-1)
        oh = (b_ref[...] == lax.broadcasted_iota(jnp.int32, (1, NG), 1))
        oh = oh.astype(jnp.float32)
        dn = (((0,), (0,)), ((), ()))
        acc[...] += lax.dot_general(oh, h64, dn, preferred_element_type=jnp.float32)
        cnt[...] += lax.dot_general(
            oh, jnp.ones((R, 1), jnp.float32), dn, preferred_element_type=jnp.float32
        )

        @pl.when(i == NB - 1)
        def _():
            pooled = acc[...] / jnp.maximum(cnt[...], 1.0)
            pp = jnp.dot(pooled, wp_ref[...], preferred_element_type=jnp.float32)
            pp = jax.nn.softplus(pp + bp_ref[...])
            o_ref[...] = (
                jnp.dot(pp, wo_ref[...], preferred_element_type=jnp.float32)
                + bo_ref[...]
            )

    full = lambda shape: pl.BlockSpec(shape, lambda i: tuple(0 for _ in shape))
    return pl.pallas_call(
        body,
        grid=(NB,),
        in_specs=[
            pl.BlockSpec((NC, R, HH), lambda i: (0, i, 0)),
            pl.BlockSpec((R, 1), lambda i: (i, 0)),
            full((H, hout)),
            full((1, hout)),
            full((hout, 1)),
            full((1, 1)),
        ],
        out_specs=pl.BlockSpec((NG, 1), lambda i: (0, 0)),
        out_shape=jax.ShapeDtypeStruct((NG, 1), jnp.float32),
        scratch_shapes=[
            pltpu.VMEM((NG, H), jnp.float32),
            pltpu.VMEM((NG, 1), jnp.float32),
        ],
    )(h2, batch_col, WprojT, bproj, WoutT, bout)


def _gate_weightT(W):
    Wt = W.T.astype(jnp.float32)
    return jnp.concatenate(
        [Wt[: H + 35], jnp.zeros((HH - 3, H), jnp.float32), Wt[H + 35 :]], axis=0
    )


def kernel(x, hyperedge_index, hyperedge_attr, batch, Wemb, bemb, Wf, bf, Wc, bc,
           Wproj, bproj, Wout, bout):
    f32 = jnp.float32
    src = hyperedge_index[0].astype(jnp.int32)
    hid = hyperedge_index[1].astype(jnp.int32)
    e = src.shape[0]
    wave = NS * CH
    ep = -(-e // wave) * wave
    nch = ep // CH
    pad = ep - e

    zi = jnp.zeros((pad,), jnp.int32)
    di = jnp.full((pad,), N, jnp.int32)
    srcg = jnp.concatenate([src, zi])
    hidg = jnp.concatenate([hid, zi])
    srcs = jnp.concatenate([src, di]).reshape(nch, CH)
    hids = jnp.concatenate([hid, di]).reshape(nch, CH)
    idx_src_g = jnp.stack([srcg, srcg + NP]).reshape(NC, nch, CH)
    idx_hid_g = jnp.stack([hidg, hidg + NP]).reshape(NC, nch, CH)
    idx_cnt = jnp.stack([srcs, hids])

    zrows = jnp.zeros((RPT, HH), f32)
    zrows16 = jnp.zeros((RPT, 16), f32)
    ones_rows = jnp.ones((CH, 16), f32)

    ap = jnp.pad(hyperedge_attr.astype(f32),
                 ((0, NP - N), (0, H - hyperedge_attr.shape[1])))
    attr_flat = jnp.stack([ap[:, :HH], ap[:, HH:]]).reshape(NC * NP, HH)

    counts = _seg_count(idx_cnt, ones_rows, zrows16)
    cnt_n = counts[0, :, :1]
    cnt_h = counts[1, :, :1]

    att2 = _seg_sum(attr_flat, idx_hid_g, srcs, zrows)

    h2 = _embed(x.astype(f32), Wemb.T.astype(f32), bemb.astype(f32).reshape(1, H))

    n_layers = Wf.shape[0]
    for l in range(n_layers):
        msum2 = _seg_sum(h2.reshape(NC * NP, HH), idx_src_g, hids, zrows)
        m2 = _scale_rows(msum2, cnt_h)
        gsum2 = _seg_sum(m2.reshape(NC * NP, HH), idx_hid_g, srcs, zrows)
        h2 = _layer(h2, att2, gsum2, cnt_n,
                    _gate_weightT(Wf[l]), bf[l].astype(f32).reshape(1, H),
                    _gate_weightT(Wc[l]), bc[l].astype(f32).reshape(1, H))

    return _pool_head(
        h2,
        batch.astype(jnp.int32).reshape(N, 1),
        Wproj.T.astype(f32),
        bproj.astype(f32).reshape(1, -1),
        Wout.T.astype(f32),
        bout.astype(f32).reshape(1, 1),
    )


# speedup vs baseline: 7.2841x; 1.8701x over previous
import functools

import jax
import jax.numpy as jnp
from jax import lax
from jax.experimental import pallas as pl
from jax.experimental.pallas import tpu as pltpu
from jax.experimental.pallas import tpu_sc as plsc

NC = 2
NS = 16
CH = 128
N = 50000
NP = 50176
RPT = NP // NS
NG = 256
HH = 32
H = 2 * HH
DB = 4
R = 2000
NB = N // R


def _sc_mesh():
    return plsc.VectorSubcoreMesh(
        core_axis_name="c", subcore_axis_name="s", num_cores=NC, num_subcores=NS
    )


def _seg_sum(table_flat, idx_g, idx_s, zrows):
    nch = idx_s.shape[0]
    cpt = nch // NS
    nw = cpt // DB

    def body(table, idxg, idxs, zr, out, acc, gbufw, sbufw, *rest):
        rows = rest[:DB]
        gsem = rest[DB : 2 * DB]
        ssem = rest[2 * DB :]
        c = lax.axis_index("c")
        s = lax.axis_index("s")
        pltpu.sync_copy(zr, acc.at[pl.ds(s * RPT, RPT)])
        plsc.subcore_barrier()

        def wave(jw, carry):
            w0 = s * cpt + jw * DB
            pltpu.sync_copy(idxg.at[c, pl.ds(w0, DB)], gbufw)
            pltpu.sync_copy(idxs.at[pl.ds(w0, DB)], sbufw)
            gds = [
                pltpu.async_copy(table.at[gbufw.at[b]], rows[b], gsem[b])
                for b in range(DB)
            ]
            sds = []
            for b in range(DB):
                gds[b].wait()
                sds.append(
                    pltpu.async_copy(
                        rows[b], acc.at[sbufw.at[b]], ssem[b], add=True
                    )
                )
            for b in range(DB):
                sds[b].wait()
            return carry

        lax.fori_loop(0, nw, wave, 0)
        plsc.subcore_barrier()
        pltpu.sync_copy(acc.at[pl.ds(s * RPT, RPT)], out.at[c, pl.ds(s * RPT, RPT)])

    return pl.kernel(
        body,
        out_type=jax.ShapeDtypeStruct((NC, NP, HH), jnp.float32),
        mesh=_sc_mesh(),
        scratch_types=[
            pltpu.VMEM_SHARED((NP, HH), jnp.float32),
            pltpu.VMEM((DB, CH), jnp.int32),
            pltpu.VMEM((DB, CH), jnp.int32),
        ]
        + [pltpu.VMEM((CH, HH), jnp.float32)] * DB
        + [pltpu.SemaphoreType.DMA] * (2 * DB),
        compiler_params=pltpu.CompilerParams(use_tc_tiling_on_sc=False),
    )(table_flat, idx_g, idx_s, zrows)


def _seg_count(idx_s2, ones_rows, zrows):
    nch = idx_s2.shape[1]
    cpt = nch // NS

    def body(idxs2, ones_h, zr, out, acc, sbuf, onesv):
        c = lax.axis_index("c")
        s = lax.axis_index("s")
        pltpu.sync_copy(zr, acc.at[pl.ds(s * RPT, RPT)])
        pltpu.sync_copy(ones_h, onesv)
        plsc.subcore_barrier()

        def step(k, carry):
            ch = s * cpt + k
            pltpu.sync_copy(idxs2.at[c, ch], sbuf)
            pltpu.sync_copy(onesv, acc.at[sbuf], add=True)
            return carry

        lax.fori_loop(0, cpt, step, 0)
        plsc.subcore_barrier()
        pltpu.sync_copy(acc.at[pl.ds(s * RPT, RPT)], out.at[c, pl.ds(s * RPT, RPT)])

    return pl.kernel(
        body,
        out_type=jax.ShapeDtypeStruct((NC, NP, 16), jnp.float32),
        mesh=_sc_mesh(),
        scratch_types=[
            pltpu.VMEM_SHARED((NP, 16), jnp.float32),
            pltpu.VMEM((CH,), jnp.int32),
            pltpu.VMEM((CH, 16), jnp.float32),
        ],
        compiler_params=pltpu.CompilerParams(use_tc_tiling_on_sc=False),
    )(idx_s2, ones_rows, zrows)


def _embed(x, WembT, bemb):
    din = x.shape[1]

    def body(x_ref, w_ref, b_ref, o_ref):
        h = jnp.dot(x_ref[...], w_ref[...], preferred_element_type=jnp.float32)
        h = h + b_ref[...]
        o_ref[0] = h[:, :HH]
        o_ref[1] = h[:, HH:]

    return pl.pallas_call(
        body,
        grid=(NB,),
        in_specs=[
            pl.BlockSpec((R, din), lambda i: (i, 0)),
            pl.BlockSpec((din, H), lambda i: (0, 0)),
            pl.BlockSpec((1, H), lambda i: (0, 0)),
        ],
        out_specs=pl.BlockSpec((NC, R, HH), lambda i: (0, i, 0)),
        out_shape=jax.ShapeDtypeStruct((NC, NP, HH), jnp.float32),
    )(x, WembT, bemb)


def _scale_rows(msum, cnt):
    def body(m_ref, c_ref, o_ref):
        inv = 1.0 / jnp.maximum(c_ref[...], 1.0)
        o_ref[...] = m_ref[...] * jnp.reshape(inv, (1, R, 1))

    return pl.pallas_call(
        body,
        grid=(NB,),
        in_specs=[
            pl.BlockSpec((NC, R, HH), lambda i: (0, i, 0)),
            pl.BlockSpec((R, 1), lambda i: (i, 0)),
        ],
        out_specs=pl.BlockSpec((NC, R, HH), lambda i: (0, i, 0)),
        out_shape=jax.ShapeDtypeStruct((NC, NP, HH), jnp.float32),
    )(msum, cnt)


def _layer(h2, a2, g2, cnt_n, WfT, bf, WcT, bc):
    def body(h_ref, a_ref, g_ref, c_ref, wf_ref, bf_ref, wc_ref, bc_ref, o_ref):
        h64 = jnp.concatenate([h_ref[0], h_ref[1]], axis=-1)
        a64 = jnp.concatenate([a_ref[0], a_ref[1]], axis=-1)
        g64 = jnp.concatenate([g_ref[0], g_ref[1]], axis=-1)
        cn = c_ref[...]
        occ = (cn > 0.0).astype(jnp.float32)
        inv = 1.0 / jnp.maximum(cn, 1.0)
        zn = jnp.concatenate([h64 * occ, a64 * inv, g64 * inv], axis=-1)
        zf = jnp.dot(zn, wf_ref[...], preferred_element_type=jnp.float32) + bf_ref[...]
        zc = jnp.dot(zn, wc_ref[...], preferred_element_type=jnp.float32) + bc_ref[...]
        outv = jax.nn.sigmoid(zf) * jax.nn.softplus(zc)
        hn = jax.nn.softplus(outv + h64)
        o_ref[0] = hn[:, :HH]
        o_ref[1] = hn[:, HH:]

    full = lambda shape: pl.BlockSpec(shape, lambda i: tuple(0 for _ in shape))
    return pl.pallas_call(
        body,
        grid=(NB,),
        in_specs=[
            pl.BlockSpec((NC, R, HH), lambda i: (0, i, 0)),
            pl.BlockSpec((NC, R, HH), lambda i: (0, i, 0)),
            pl.BlockSpec((NC, R, HH), lambda i: (0, i, 0)),
            pl.BlockSpec((R, 1), lambda i: (i, 0)),
            full((3 * H, H)),
            full((1, H)),
            full((3 * H, H)),
            full((1, H)),
        ],
        out_specs=pl.BlockSpec((NC, R, HH), lambda i: (0, i, 0)),
        out_shape=jax.ShapeDtypeStruct((NC, NP, HH), jnp.float32),
    )(h2, a2, g2, cnt_n, WfT, bf, WcT, bc)


def _pool_head(h2, batch_col, WprojT, bproj, WoutT, bout):
    hout = WprojT.shape[1]

    def body(h_ref, b_ref, wp_ref, bp_ref, wo_ref, bo_ref, o_ref, acc, cnt):
        i = pl.program_id(0)

        @pl.when(i == 0)
        def _():
            acc[...] = jnp.zeros_like(acc)
            cnt[...] = jnp.zeros_like(cnt)

        h64 = jnp.concatenate([h_ref[0], h_ref[1]], axis=-1)
        oh = (b_ref[...] == lax.broadcasted_iota(jnp.int32, (1, NG), 1))
        oh = oh.astype(jnp.float32)
        dn = (((0,), (0,)), ((), ()))
        acc[...] += lax.dot_general(oh, h64, dn, preferred_element_type=jnp.float32)
        cnt[...] += lax.dot_general(
            oh, jnp.ones((R, 1), jnp.float32), dn, preferred_element_type=jnp.float32
        )

        @pl.when(i == NB - 1)
        def _():
            pooled = acc[...] / jnp.maximum(cnt[...], 1.0)
            pp = jnp.dot(pooled, wp_ref[...], preferred_element_type=jnp.float32)
            pp = jax.nn.softplus(pp + bp_ref[...])
            o_ref[...] = (
                jnp.dot(pp, wo_ref[...], preferred_element_type=jnp.float32)
                + bo_ref[...]
            )

    full = lambda shape: pl.BlockSpec(shape, lambda i: tuple(0 for _ in shape))
    return pl.pallas_call(
        body,
        grid=(NB,),
        in_specs=[
            pl.BlockSpec((NC, R, HH), lambda i: (0, i, 0)),
            pl.BlockSpec((R, 1), lambda i: (i, 0)),
            full((H, hout)),
            full((1, hout)),
            full((hout, 1)),
            full((1, 1)),
        ],
        out_specs=pl.BlockSpec((NG, 1), lambda i: (0, 0)),
        out_shape=jax.ShapeDtypeStruct((NG, 1), jnp.float32),
        scratch_shapes=[
            pltpu.VMEM((NG, H), jnp.float32),
            pltpu.VMEM((NG, 1), jnp.float32),
        ],
    )(h2, batch_col, WprojT, bproj, WoutT, bout)


def _gate_weightT(W):
    Wt = W.T.astype(jnp.float32)
    return jnp.concatenate(
        [Wt[: H + 35], jnp.zeros((HH - 3, H), jnp.float32), Wt[H + 35 :]], axis=0
    )


def kernel(x, hyperedge_index, hyperedge_attr, batch, Wemb, bemb, Wf, bf, Wc, bc,
           Wproj, bproj, Wout, bout):
    f32 = jnp.float32
    src = hyperedge_index[0].astype(jnp.int32)
    hid = hyperedge_index[1].astype(jnp.int32)
    e = src.shape[0]
    wave = NS * CH * DB
    ep = -(-e // wave) * wave
    nch = ep // CH
    pad = ep - e

    zi = jnp.zeros((pad,), jnp.int32)
    di = jnp.full((pad,), N, jnp.int32)
    srcg = jnp.concatenate([src, zi])
    hidg = jnp.concatenate([hid, zi])
    srcs = jnp.concatenate([src, di]).reshape(nch, CH)
    hids = jnp.concatenate([hid, di]).reshape(nch, CH)
    idx_src_g = jnp.stack([srcg, srcg + NP]).reshape(NC, nch, CH)
    idx_hid_g = jnp.stack([hidg, hidg + NP]).reshape(NC, nch, CH)
    idx_cnt = jnp.stack([srcs, hids])

    zrows = jnp.zeros((RPT, HH), f32)
    zrows16 = jnp.zeros((RPT, 16), f32)
    ones_rows = jnp.ones((CH, 16), f32)

    ap = jnp.pad(hyperedge_attr.astype(f32),
                 ((0, NP - N), (0, H - hyperedge_attr.shape[1])))
    attr_flat = jnp.stack([ap[:, :HH], ap[:, HH:]]).reshape(NC * NP, HH)

    counts = _seg_count(idx_cnt, ones_rows, zrows16)
    cnt_n = counts[0, :, :1]
    cnt_h = counts[1, :, :1]

    att2 = _seg_sum(attr_flat, idx_hid_g, srcs, zrows)

    h2 = _embed(x.astype(f32), Wemb.T.astype(f32), bemb.astype(f32).reshape(1, H))

    n_layers = Wf.shape[0]
    for l in range(n_layers):
        msum2 = _seg_sum(h2.reshape(NC * NP, HH), idx_src_g, hids, zrows)
        m2 = _scale_rows(msum2, cnt_h)
        gsum2 = _seg_sum(m2.reshape(NC * NP, HH), idx_hid_g, srcs, zrows)
        h2 = _layer(h2, att2, gsum2, cnt_n,
                    _gate_weightT(Wf[l]), bf[l].astype(f32).reshape(1, H),
                    _gate_weightT(Wc[l]), bc[l].astype(f32).reshape(1, H))

    return _pool_head(
        h2,
        batch.astype(jnp.int32).reshape(N, 1),
        Wproj.T.astype(f32),
        bproj.astype(f32).reshape(1, -1),
        Wout.T.astype(f32),
        bout.astype(f32).reshape(1, 1),
    )
